# baseline (speedup 1.0000x reference)
import functools

import jax
import jax.numpy as jnp
from jax import lax
from jax.experimental import pallas as pl
from jax.experimental.pallas import tpu as pltpu
from jax.experimental.pallas import tpu_sc as plsc

N = 5000
C = 80
THR = 0.2
IOU_THR = 0.2
K = 100

L = 16
NS = 16
ROWS = 320
NV = ROWS // L
NP = NS * ROWS
NEG = -1e30
BIG = 1 << 30
BO_PAD = 448
SC_PAD = 112


def _nms_kernel(sc_hbm, x1_hbm, y1_hbm, x2_hbm, y2_hbm,
                bo_hbm, so_hbm, lo_hbm,
                scv, x1v, y1v, x2v, y2v, arv, candv, labv,
                pubv, rbv, obv, osv, olv, shared):
    sid = lax.axis_index("s")
    base = sid * ROWS
    lanes = lax.iota(jnp.int32, L)
    zf = jnp.zeros((L,), jnp.float32)
    zi = jnp.zeros((L,), jnp.int32)
    negv = jnp.full((L,), NEG, jnp.float32)
    bigv = jnp.full((L,), BIG, jnp.int32)

    pltpu.sync_copy(sc_hbm.at[pl.ds(base * C, ROWS * C)], scv)
    pltpu.sync_copy(x1_hbm.at[pl.ds(base, ROWS)], x1v)
    pltpu.sync_copy(y1_hbm.at[pl.ds(base, ROWS)], y1v)
    pltpu.sync_copy(x2_hbm.at[pl.ds(base, ROWS)], x2v)
    pltpu.sync_copy(y2_hbm.at[pl.ds(base, ROWS)], y2v)

    def zero_b(j, _):
        obv[pl.ds(j * L, L)] = zf
        return 0
    lax.fori_loop(0, BO_PAD // L, zero_b, 0)

    def zero_s(j, _):
        osv[pl.ds(j * L, L)] = zf
        olv[pl.ds(j * L, L)] = zi
        return 0
    lax.fori_loop(0, SC_PAD // L, zero_s, 0)

    def grp(g, bSI):
        bS, bI = bSI
        row_off = g * (L * C) + lanes * C

        def cls(c8, MA):
            M, A = MA
            for dc in range(8):
                c = c8 * 8 + dc
                v = plsc.load_gather(scv, [row_off + c])
                take = v > M
                M = jnp.where(take, v, M)
                A = jnp.where(take, jnp.full((L,), c, jnp.int32), A)
            return (M, A)

        M, A = lax.fori_loop(0, C // 8, cls,
                             (jnp.full((L,), NEG, jnp.float32), zi))
        sl = pl.ds(g * L, L)
        cand = jnp.where(M > THR, M, negv)
        candv[sl] = cand
        labv[sl] = A
        arv[sl] = (x2v[sl] - x1v[sl]) * (y2v[sl] - y1v[sl])
        take = cand > bS
        return (jnp.where(take, cand, bS),
                jnp.where(take, g * L + lanes, bI))

    bS0, bI0 = lax.fori_loop(
        0, NV, grp,
        (jnp.full((L,), NEG, jnp.float32), bigv))

    def cond(st):
        return st[1]

    def body(st):
        k, _, bS, bI = st

        Ml = jnp.max(bS)
        li = jnp.min(jnp.where(bS == Ml, bI, bigv))
        lic = jnp.clip(li, 0, ROWS - 1)
        liv = jnp.full((L,), lic, jnp.int32)
        cx1 = plsc.load_gather(x1v, [liv])
        cy1 = plsc.load_gather(y1v, [liv])
        cx2 = plsc.load_gather(x2v, [liv])
        cy2 = plsc.load_gather(y2v, [liv])
        car = plsc.load_gather(arv, [liv])
        clb = plsc.load_gather(labv, [liv])

        pub = jnp.full((L,), Ml, jnp.float32)
        packed = jnp.full((L,), sid * 65536 + lic, jnp.int32)
        pub = jnp.where(lanes == 1, plsc.bitcast(packed, jnp.float32), pub)
        pub = jnp.where(lanes == 2, cx1, pub)
        pub = jnp.where(lanes == 3, cy1, pub)
        pub = jnp.where(lanes == 4, cx2, pub)
        pub = jnp.where(lanes == 5, cy2, pub)
        pub = jnp.where(lanes == 6, plsc.bitcast(clb, jnp.float32), pub)
        pub = jnp.where(lanes == 7, car, pub)
        pubv[...] = pub

        par = lax.rem(k, 2)
        pltpu.sync_copy(pubv, shared.at[pl.ds(par * (NS * L) + sid * L, L)])
        plsc.subcore_barrier()
        pltpu.sync_copy(shared.at[pl.ds(par * (NS * L), NS * L)], rbv)

        s_all = plsc.load_gather(rbv, [lanes * L])
        p_all = plsc.bitcast(plsc.load_gather(rbv, [lanes * L + 1]),
                             jnp.int32)
        M = jnp.max(s_all)
        cont2 = M > -1e29
        iwp = jnp.min(jnp.where(s_all == M, p_all, bigv))
        wsid = jnp.right_shift(iwp, 16)
        wrow = jnp.clip(wsid, 0, NS - 1) * L
        iwv = jnp.full((L,), (iwp & 0xFFFF) + wsid * ROWS, jnp.int32)

        def slot(c):
            return plsc.load_gather(rbv, [jnp.full((L,), wrow + c,
                                                   jnp.int32)])

        X1v = slot(2)
        Y1v = slot(3)
        X2v = slot(4)
        Y2v = slot(5)
        LBv = plsc.bitcast(slot(6), jnp.int32)
        WAv = slot(7)

        def sweep(j4, nSI):
            nS, nI = nSI
            for dj in range(4):
                j = j4 * 4 + dj
                sl = pl.ds(j * L, L)
                xx1 = jnp.maximum(x1v[sl], X1v)
                yy1 = jnp.maximum(y1v[sl], Y1v)
                xx2 = jnp.minimum(x2v[sl], X2v)
                yy2 = jnp.minimum(y2v[sl], Y2v)
                inter = (jnp.maximum(xx2 - xx1, 0.0) *
                         jnp.maximum(yy2 - yy1, 0.0))
                union = WAv + arv[sl] - inter
                iou = inter / (union + 1e-8)
                gi = base + j * L + lanes
                kill = (iou >= IOU_THR) | (gi == iwv)
                nc = jnp.where(kill, negv, candv[sl])
                candv[sl] = nc
                take = nc > nS
                nS = jnp.where(take, nc, nS)
                nI = jnp.where(take, jnp.full((L,), j * L, jnp.int32) + lanes,
                               nI)
            return (nS, nI)

        nS, nI = lax.fori_loop(
            0, NV // 4, sweep,
            (jnp.full((L,), NEG, jnp.float32), bigv))

        @pl.when(cont2 & (sid == 0))
        def _():
            bvals = X1v
            bvals = jnp.where(lanes == 1, Y1v, bvals)
            bvals = jnp.where(lanes == 2, X2v, bvals)
            bvals = jnp.where(lanes == 3, Y2v, bvals)
            plsc.store_scatter(obv, [4 * k + lanes], bvals, mask=lanes < 4)
            kv = jnp.full((L,), k, jnp.int32)
            plsc.store_scatter(osv, [kv], jnp.full((L,), M, jnp.float32),
                               mask=lanes == 0)
            plsc.store_scatter(olv, [kv], LBv, mask=lanes == 0)

        k2 = k + cont2.astype(jnp.int32)
        return (k2, cont2 & (k2 < K), nS, nI)

    lax.while_loop(cond, body, (jnp.int32(0), jnp.bool_(True), bS0, bI0))

    @pl.when(sid == 0)
    def _():
        pltpu.sync_copy(obv, bo_hbm)
        pltpu.sync_copy(osv, so_hbm)
        pltpu.sync_copy(olv, lo_hbm)


@functools.partial(
    pl.kernel,
    out_type=(
        jax.ShapeDtypeStruct((BO_PAD,), jnp.float32),
        jax.ShapeDtypeStruct((SC_PAD,), jnp.float32),
        jax.ShapeDtypeStruct((SC_PAD,), jnp.int32),
    ),
    mesh=plsc.VectorSubcoreMesh(
        core_axis_name="c", subcore_axis_name="s",
        num_cores=1, num_subcores=NS),
    compiler_params=pltpu.CompilerParams(needs_layout_passes=False),
    scratch_types=[
        pltpu.VMEM((ROWS * C,), jnp.float32),
        pltpu.VMEM((ROWS,), jnp.float32),
        pltpu.VMEM((ROWS,), jnp.float32),
        pltpu.VMEM((ROWS,), jnp.float32),
        pltpu.VMEM((ROWS,), jnp.float32),
        pltpu.VMEM((ROWS,), jnp.float32),
        pltpu.VMEM((ROWS,), jnp.float32),
        pltpu.VMEM((ROWS,), jnp.int32),
        pltpu.VMEM((L,), jnp.float32),
        pltpu.VMEM((NS * L,), jnp.float32),
        pltpu.VMEM((BO_PAD,), jnp.float32),
        pltpu.VMEM((SC_PAD,), jnp.float32),
        pltpu.VMEM((SC_PAD,), jnp.int32),
        pltpu.VMEM_SHARED((2 * NS * L,), jnp.float32),
    ],
)
def _nms_call(sc_hbm, x1_hbm, y1_hbm, x2_hbm, y2_hbm,
              bo_hbm, so_hbm, lo_hbm, *scratch):
    _nms_kernel(sc_hbm, x1_hbm, y1_hbm, x2_hbm, y2_hbm,
                bo_hbm, so_hbm, lo_hbm, *scratch)


@jax.jit
def kernel(boxes, scores):
    bp = jnp.pad(boxes, ((0, NP - N), (0, 0)))
    sp = jnp.pad(scores, ((0, NP - N), (0, 0)))
    bo, so, lo = _nms_call(
        sp.reshape(-1),
        bp[:, 0], bp[:, 1], bp[:, 2], bp[:, 3])
    return (bo[:4 * K].reshape(1, K, 4), so[:K][None], lo[:K][None])


# speedup vs baseline: 1.0047x; 1.0047x over previous
import functools

import jax
import jax.numpy as jnp
from jax import lax
from jax.experimental import pallas as pl
from jax.experimental.pallas import tpu as pltpu
from jax.experimental.pallas import tpu_sc as plsc

N = 5000
C = 80
THR = 0.2
IOU_THR = 0.2
K = 100

L = 16
NS = 16
ROWS = 320
NV = ROWS // L
NP = NS * ROWS
NEG = -1e30
BIG = 1 << 30
BO_PAD = 448
SC_PAD = 112


def _nms_kernel(sc_hbm, x1_hbm, y1_hbm, x2_hbm, y2_hbm,
                bo_hbm, so_hbm, lo_hbm,
                scv, x1v, y1v, x2v, y2v, arv, candv, labv,
                pubv, rbv, obv, osv, olv, shared):
    sid = lax.axis_index("s")
    base = sid * ROWS
    lanes = lax.iota(jnp.int32, L)
    zf = jnp.zeros((L,), jnp.float32)
    zi = jnp.zeros((L,), jnp.int32)
    negv = jnp.full((L,), NEG, jnp.float32)
    bigv = jnp.full((L,), BIG, jnp.int32)

    @pl.when(sid < NS - 1)
    def _():
        pltpu.sync_copy(sc_hbm.at[pl.ds(base * C, ROWS * C)], scv)

    @pl.when(sid == NS - 1)
    def _():
        pltpu.sync_copy(sc_hbm.at[pl.ds(base * C, (N - (NS - 1) * ROWS) * C)],
                        scv.at[pl.ds(0, (N - (NS - 1) * ROWS) * C)])
    pltpu.sync_copy(x1_hbm.at[pl.ds(base, ROWS)], x1v)
    pltpu.sync_copy(y1_hbm.at[pl.ds(base, ROWS)], y1v)
    pltpu.sync_copy(x2_hbm.at[pl.ds(base, ROWS)], x2v)
    pltpu.sync_copy(y2_hbm.at[pl.ds(base, ROWS)], y2v)

    def zero_b(j, _):
        obv[pl.ds(j * L, L)] = zf
        return 0
    lax.fori_loop(0, BO_PAD // L, zero_b, 0)

    def zero_s(j, _):
        osv[pl.ds(j * L, L)] = zf
        olv[pl.ds(j * L, L)] = zi
        return 0
    lax.fori_loop(0, SC_PAD // L, zero_s, 0)

    def grp(g, bSI):
        bS, bI = bSI
        row_off = g * (L * C) + lanes * C

        def cls(c8, MA):
            M, A = MA
            for dc in range(8):
                c = c8 * 8 + dc
                v = plsc.load_gather(scv, [row_off + c])
                take = v > M
                M = jnp.where(take, v, M)
                A = jnp.where(take, jnp.full((L,), c, jnp.int32), A)
            return (M, A)

        M, A = lax.fori_loop(0, C // 8, cls,
                             (jnp.full((L,), NEG, jnp.float32), zi))
        sl = pl.ds(g * L, L)
        validrow = (base + g * L + lanes) < N
        cand = jnp.where(validrow & (M > THR), M, negv)
        candv[sl] = cand
        labv[sl] = A
        arv[sl] = (x2v[sl] - x1v[sl]) * (y2v[sl] - y1v[sl])
        take = cand > bS
        return (jnp.where(take, cand, bS),
                jnp.where(take, g * L + lanes, bI))

    bS0, bI0 = lax.fori_loop(
        0, NV, grp,
        (jnp.full((L,), NEG, jnp.float32), bigv))

    def cond(st):
        return st[1]

    def body(st):
        k, _, bS, bI = st

        Ml = jnp.max(bS)
        li = jnp.min(jnp.where(bS == Ml, bI, bigv))
        lic = jnp.clip(li, 0, ROWS - 1)
        liv = jnp.full((L,), lic, jnp.int32)
        cx1 = plsc.load_gather(x1v, [liv])
        cy1 = plsc.load_gather(y1v, [liv])
        cx2 = plsc.load_gather(x2v, [liv])
        cy2 = plsc.load_gather(y2v, [liv])
        car = plsc.load_gather(arv, [liv])
        clb = plsc.load_gather(labv, [liv])

        pub = jnp.full((L,), Ml, jnp.float32)
        packed = jnp.full((L,), sid * 65536 + lic, jnp.int32)
        pub = jnp.where(lanes == 1, plsc.bitcast(packed, jnp.float32), pub)
        pub = jnp.where(lanes == 2, cx1, pub)
        pub = jnp.where(lanes == 3, cy1, pub)
        pub = jnp.where(lanes == 4, cx2, pub)
        pub = jnp.where(lanes == 5, cy2, pub)
        pub = jnp.where(lanes == 6, plsc.bitcast(clb, jnp.float32), pub)
        pub = jnp.where(lanes == 7, car, pub)
        pubv[...] = pub

        par = lax.rem(k, 2)
        pltpu.sync_copy(pubv, shared.at[pl.ds(par * (NS * L) + sid * L, L)])
        plsc.subcore_barrier()
        pltpu.sync_copy(shared.at[pl.ds(par * (NS * L), NS * L)], rbv)

        s_all = plsc.load_gather(rbv, [lanes * L])
        p_all = plsc.bitcast(plsc.load_gather(rbv, [lanes * L + 1]),
                             jnp.int32)
        M = jnp.max(s_all)
        cont2 = M > -1e29
        iwp = jnp.min(jnp.where(s_all == M, p_all, bigv))
        wsid = jnp.right_shift(iwp, 16)
        wrow = jnp.clip(wsid, 0, NS - 1) * L
        iwv = jnp.full((L,), (iwp & 0xFFFF) + wsid * ROWS, jnp.int32)

        def slot(c):
            return plsc.load_gather(rbv, [jnp.full((L,), wrow + c,
                                                   jnp.int32)])

        X1v = slot(2)
        Y1v = slot(3)
        X2v = slot(4)
        Y2v = slot(5)
        LBv = plsc.bitcast(slot(6), jnp.int32)
        WAv = slot(7)

        def sweep(j4, nSI):
            nS, nI = nSI
            for dj in range(4):
                j = j4 * 4 + dj
                sl = pl.ds(j * L, L)
                xx1 = jnp.maximum(x1v[sl], X1v)
                yy1 = jnp.maximum(y1v[sl], Y1v)
                xx2 = jnp.minimum(x2v[sl], X2v)
                yy2 = jnp.minimum(y2v[sl], Y2v)
                inter = (jnp.maximum(xx2 - xx1, 0.0) *
                         jnp.maximum(yy2 - yy1, 0.0))
                union = WAv + arv[sl] - inter
                iou = inter / (union + 1e-8)
                gi = base + j * L + lanes
                kill = (iou >= IOU_THR) | (gi == iwv)
                nc = jnp.where(kill, negv, candv[sl])
                candv[sl] = nc
                take = nc > nS
                nS = jnp.where(take, nc, nS)
                nI = jnp.where(take, jnp.full((L,), j * L, jnp.int32) + lanes,
                               nI)
            return (nS, nI)

        nS, nI = lax.fori_loop(
            0, NV // 4, sweep,
            (jnp.full((L,), NEG, jnp.float32), bigv))

        @pl.when(cont2 & (sid == 0))
        def _():
            bvals = X1v
            bvals = jnp.where(lanes == 1, Y1v, bvals)
            bvals = jnp.where(lanes == 2, X2v, bvals)
            bvals = jnp.where(lanes == 3, Y2v, bvals)
            plsc.store_scatter(obv, [4 * k + lanes], bvals, mask=lanes < 4)
            kv = jnp.full((L,), k, jnp.int32)
            plsc.store_scatter(osv, [kv], jnp.full((L,), M, jnp.float32),
                               mask=lanes == 0)
            plsc.store_scatter(olv, [kv], LBv, mask=lanes == 0)

        k2 = k + cont2.astype(jnp.int32)
        return (k2, cont2 & (k2 < K), nS, nI)

    lax.while_loop(cond, body, (jnp.int32(0), jnp.bool_(True), bS0, bI0))

    @pl.when(sid == 0)
    def _():
        pltpu.sync_copy(obv, bo_hbm)
        pltpu.sync_copy(osv, so_hbm)
        pltpu.sync_copy(olv, lo_hbm)


@functools.partial(
    pl.kernel,
    out_type=(
        jax.ShapeDtypeStruct((BO_PAD,), jnp.float32),
        jax.ShapeDtypeStruct((SC_PAD,), jnp.float32),
        jax.ShapeDtypeStruct((SC_PAD,), jnp.int32),
    ),
    mesh=plsc.VectorSubcoreMesh(
        core_axis_name="c", subcore_axis_name="s",
        num_cores=1, num_subcores=NS),
    compiler_params=pltpu.CompilerParams(needs_layout_passes=False),
    scratch_types=[
        pltpu.VMEM((ROWS * C,), jnp.float32),
        pltpu.VMEM((ROWS,), jnp.float32),
        pltpu.VMEM((ROWS,), jnp.float32),
        pltpu.VMEM((ROWS,), jnp.float32),
        pltpu.VMEM((ROWS,), jnp.float32),
        pltpu.VMEM((ROWS,), jnp.float32),
        pltpu.VMEM((ROWS,), jnp.float32),
        pltpu.VMEM((ROWS,), jnp.int32),
        pltpu.VMEM((L,), jnp.float32),
        pltpu.VMEM((NS * L,), jnp.float32),
        pltpu.VMEM((BO_PAD,), jnp.float32),
        pltpu.VMEM((SC_PAD,), jnp.float32),
        pltpu.VMEM((SC_PAD,), jnp.int32),
        pltpu.VMEM_SHARED((2 * NS * L,), jnp.float32),
    ],
)
def _nms_call(sc_hbm, x1_hbm, y1_hbm, x2_hbm, y2_hbm,
              bo_hbm, so_hbm, lo_hbm, *scratch):
    _nms_kernel(sc_hbm, x1_hbm, y1_hbm, x2_hbm, y2_hbm,
                bo_hbm, so_hbm, lo_hbm, *scratch)


@jax.jit
def kernel(boxes, scores):
    bp = jnp.pad(boxes, ((0, NP - N), (0, 0)))
    bo, so, lo = _nms_call(
        scores.reshape(-1),
        bp[:, 0], bp[:, 1], bp[:, 2], bp[:, 3])
    return (bo[:4 * K].reshape(1, K, 4), so[:K][None], lo[:K][None])


# speedup vs baseline: 1.1395x; 1.1342x over previous
import functools

import jax
import jax.numpy as jnp
from jax import lax
from jax.experimental import pallas as pl
from jax.experimental.pallas import tpu as pltpu
from jax.experimental.pallas import tpu_sc as plsc

N = 5000
C = 80
THR = 0.2
IOU_THR = 0.2
K = 100

L = 16
NS = 16
ROWS = 320
NV = ROWS // L
NP = NS * ROWS
NEG = -1e30
BIG = 1 << 30
BO_PAD = 448
SC_PAD = 112


def _nms_kernel(sc_hbm, x1_hbm, y1_hbm, x2_hbm, y2_hbm,
                bo_hbm, so_hbm, lo_hbm,
                scv, x1v, y1v, x2v, y2v, arv, candv, labv,
                pubv, rbv, obv, osv, olv, shared):
    sid = lax.axis_index("s")
    base = sid * ROWS
    lanes = lax.iota(jnp.int32, L)
    zf = jnp.zeros((L,), jnp.float32)
    zi = jnp.zeros((L,), jnp.int32)
    negv = jnp.full((L,), NEG, jnp.float32)
    bigv = jnp.full((L,), BIG, jnp.int32)

    @pl.when(sid < NS - 1)
    def _():
        pltpu.sync_copy(sc_hbm.at[pl.ds(base * C, ROWS * C)], scv)

    @pl.when(sid == NS - 1)
    def _():
        pltpu.sync_copy(sc_hbm.at[pl.ds(base * C, (N - (NS - 1) * ROWS) * C)],
                        scv.at[pl.ds(0, (N - (NS - 1) * ROWS) * C)])

    pltpu.sync_copy(x1_hbm.at[pl.ds(base, ROWS)], x1v)
    pltpu.sync_copy(y1_hbm.at[pl.ds(base, ROWS)], y1v)
    pltpu.sync_copy(x2_hbm.at[pl.ds(base, ROWS)], x2v)
    pltpu.sync_copy(y2_hbm.at[pl.ds(base, ROWS)], y2v)

    def zero_b(j, _):
        obv[pl.ds(j * L, L)] = zf
        return 0
    lax.fori_loop(0, BO_PAD // L, zero_b, 0)

    def zero_s(j, _):
        osv[pl.ds(j * L, L)] = zf
        olv[pl.ds(j * L, L)] = zi
        return 0
    lax.fori_loop(0, SC_PAD // L, zero_s, 0)

    def top2_update(st, v, idx):
        bS1, bI1, bS2, bI2 = st
        take1 = v > bS1
        take2 = (v > bS2) & (~take1)
        nS2 = jnp.where(take1, bS1, jnp.where(take2, v, bS2))
        nI2 = jnp.where(take1, bI1, jnp.where(take2, idx, bI2))
        nS1 = jnp.where(take1, v, bS1)
        nI1 = jnp.where(take1, idx, bI1)
        return (nS1, nI1, nS2, nI2)

    def grp(g, st):
        row_off = g * (L * C) + lanes * C

        def cls(c8, MA):
            M, A = MA
            for dc in range(8):
                c = c8 * 8 + dc
                v = plsc.load_gather(scv, [row_off + c])
                take = v > M
                M = jnp.where(take, v, M)
                A = jnp.where(take, jnp.full((L,), c, jnp.int32), A)
            return (M, A)

        M, A = lax.fori_loop(0, C // 8, cls,
                             (jnp.full((L,), NEG, jnp.float32), zi))
        sl = pl.ds(g * L, L)
        validrow = (base + g * L + lanes) < N
        cand = jnp.where(validrow & (M > THR), M, negv)
        candv[sl] = cand
        labv[sl] = A
        arv[sl] = (x2v[sl] - x1v[sl]) * (y2v[sl] - y1v[sl])
        return top2_update(st, cand, g * L + lanes)

    top0 = lax.fori_loop(
        0, NV, grp,
        (jnp.full((L,), NEG, jnp.float32), bigv,
         jnp.full((L,), NEG, jnp.float32), bigv))

    def cond(st):
        return st[1]

    def body(st):
        k0, _, bS1, bI1, bS2, bI2 = st

        M1 = jnp.max(bS1)
        li1 = jnp.min(jnp.where(bS1 == M1, bI1, bigv))
        match = (bS1 == M1) & (bI1 == li1)
        bS1m = jnp.where(match, bS2, bS1)
        bI1m = jnp.where(match, bI2, bI1)
        M2 = jnp.max(bS1m)
        li2 = jnp.min(jnp.where(bS1m == M2, bI1m, bigv))
        li1c = jnp.clip(li1, 0, ROWS - 1)
        li2c = jnp.clip(li2, 0, ROWS - 1)

        l1v = jnp.full((L,), li1c, jnp.int32)
        l2v = jnp.full((L,), li2c, jnp.int32)
        a_x1 = plsc.load_gather(x1v, [l1v])
        a_y1 = plsc.load_gather(y1v, [l1v])
        a_x2 = plsc.load_gather(x2v, [l1v])
        a_y2 = plsc.load_gather(y2v, [l1v])
        a_ar = plsc.load_gather(arv, [l1v])
        a_lb = plsc.load_gather(labv, [l1v])
        b_x1 = plsc.load_gather(x1v, [l2v])
        b_y1 = plsc.load_gather(y1v, [l2v])
        b_x2 = plsc.load_gather(x2v, [l2v])
        b_y2 = plsc.load_gather(y2v, [l2v])
        b_ar = plsc.load_gather(arv, [l2v])
        b_lb = plsc.load_gather(labv, [l2v])

        pub = jnp.full((L,), M1, jnp.float32)
        pk1 = jnp.full((L,), sid * 65536 + li1c, jnp.int32)
        pk2 = jnp.full((L,), sid * 65536 + li2c, jnp.int32)
        pub = jnp.where(lanes == 1, plsc.bitcast(pk1, jnp.float32), pub)
        pub = jnp.where(lanes == 2, a_x1, pub)
        pub = jnp.where(lanes == 3, a_y1, pub)
        pub = jnp.where(lanes == 4, a_x2, pub)
        pub = jnp.where(lanes == 5, a_y2, pub)
        pub = jnp.where(lanes == 6, plsc.bitcast(a_lb, jnp.float32), pub)
        pub = jnp.where(lanes == 7, a_ar, pub)
        pub = jnp.where(lanes == 8, jnp.full((L,), M2, jnp.float32), pub)
        pub = jnp.where(lanes == 9, plsc.bitcast(pk2, jnp.float32), pub)
        pub = jnp.where(lanes == 10, b_x1, pub)
        pub = jnp.where(lanes == 11, b_y1, pub)
        pub = jnp.where(lanes == 12, b_x2, pub)
        pub = jnp.where(lanes == 13, b_y2, pub)
        pub = jnp.where(lanes == 14, plsc.bitcast(b_lb, jnp.float32), pub)
        pub = jnp.where(lanes == 15, b_ar, pub)
        pubv[...] = pub

        par = lax.rem(k0, 2)
        pltpu.sync_copy(pubv, shared.at[pl.ds(par * (NS * L) + sid * L, L)])
        plsc.subcore_barrier()
        pltpu.sync_copy(shared.at[pl.ds(par * (NS * L), NS * L)], rbv)

        def col(c):
            return plsc.load_gather(rbv, [lanes * L + c])

        s_a = col(0)
        p_a = plsc.bitcast(col(1), jnp.int32)
        ax1 = col(2)
        ay1 = col(3)
        ax2 = col(4)
        ay2 = col(5)
        alb = plsc.bitcast(col(6), jnp.int32)
        aar = col(7)
        s_b = col(8)
        p_b = plsc.bitcast(col(9), jnp.int32)
        bx1 = col(10)
        by1 = col(11)
        bx2 = col(12)
        by2 = col(13)
        blb = plsc.bitcast(col(14), jnp.int32)
        bar = col(15)

        dead_a0 = ~(s_a > -1e29)
        dead_b0 = ~(s_b > -1e29)

        def ex_cond(est):
            return est[0]

        def ex_body(est):
            _, k, dead_a, dead_b, bS1, bI1, bS2, bI2 = est

            ma = jnp.where(dead_a, negv, s_a)
            mb = jnp.where(dead_b, negv, s_b)
            M = jnp.max(jnp.maximum(ma, mb))
            Mv = jnp.full((L,), M, jnp.float32)
            pa = jnp.where((~dead_a) & (s_a == Mv), p_a, bigv)
            pb = jnp.where((~dead_b) & (s_b == Mv), p_b, bigv)
            pw = jnp.min(jnp.minimum(pa, pb))
            pwv = jnp.full((L,), pw, jnp.int32)
            got = M > -1e29
            bothdead = dead_a & dead_b
            outrank = (s_b > Mv) | ((s_b == Mv) & (p_b < pwv))
            viol = jnp.max((bothdead & outrank).astype(jnp.int32)) > 0
            accept = got & (~viol) & (k < K)

            is_w_a = (~dead_a) & (s_a == Mv) & (p_a == pwv)
            is_w_b = (~dead_b) & (s_b == Mv) & (p_b == pwv)

            wx1 = jnp.max(jnp.where(is_w_a, ax1, jnp.where(is_w_b, bx1, negv)))
            wy1 = jnp.max(jnp.where(is_w_a, ay1, jnp.where(is_w_b, by1, negv)))
            wx2 = jnp.max(jnp.where(is_w_a, ax2, jnp.where(is_w_b, bx2, negv)))
            wy2 = jnp.max(jnp.where(is_w_a, ay2, jnp.where(is_w_b, by2, negv)))
            war = jnp.max(jnp.where(is_w_a, aar, jnp.where(is_w_b, bar, negv)))
            wlb = jnp.max(jnp.where(is_w_a, alb,
                                    jnp.where(is_w_b, blb,
                                              jnp.full((L,), -1, jnp.int32))))

            X1 = jnp.where(accept, wx1, NEG)
            Y1 = jnp.where(accept, wy1, NEG)
            X2 = jnp.where(accept, wx2, NEG)
            Y2 = jnp.where(accept, wy2, NEG)
            WA = jnp.where(accept, war, NEG)
            iw = jnp.where(accept, (pw & 0xFFFF)
                           + jnp.right_shift(pw, 16) * ROWS, -1)
            X1v = jnp.full((L,), X1, jnp.float32)
            Y1v = jnp.full((L,), Y1, jnp.float32)
            X2v = jnp.full((L,), X2, jnp.float32)
            Y2v = jnp.full((L,), Y2, jnp.float32)
            WAv = jnp.full((L,), WA, jnp.float32)
            iwv = jnp.full((L,), iw, jnp.int32)
            acc_v = jnp.full((L,), accept, jnp.bool_)

            def board_iou(ex1, ey1, ex2, ey2, ear):
                xx1 = jnp.maximum(ex1, X1v)
                yy1 = jnp.maximum(ey1, Y1v)
                xx2 = jnp.minimum(ex2, X2v)
                yy2 = jnp.minimum(ey2, Y2v)
                inter = (jnp.maximum(xx2 - xx1, 0.0) *
                         jnp.maximum(yy2 - yy1, 0.0))
                union = WAv + ear - inter
                return inter / (union + 1e-8)

            kill_a = acc_v & (is_w_a | (board_iou(ax1, ay1, ax2, ay2, aar)
                                        >= IOU_THR))
            kill_b = acc_v & (is_w_b | (board_iou(bx1, by1, bx2, by2, bar)
                                        >= IOU_THR))
            dead_a = dead_a | kill_a
            dead_b = dead_b | kill_b

            def sweep(j4, nst):
                for dj in range(4):
                    j = j4 * 4 + dj
                    sl = pl.ds(j * L, L)
                    xx1 = jnp.maximum(x1v[sl], X1v)
                    yy1 = jnp.maximum(y1v[sl], Y1v)
                    xx2 = jnp.minimum(x2v[sl], X2v)
                    yy2 = jnp.minimum(y2v[sl], Y2v)
                    inter = (jnp.maximum(xx2 - xx1, 0.0) *
                             jnp.maximum(yy2 - yy1, 0.0))
                    union = WAv + arv[sl] - inter
                    iou = inter / (union + 1e-8)
                    gi = base + j * L + lanes
                    kill = (iou >= IOU_THR) | (gi == iwv)
                    nc = jnp.where(kill, negv, candv[sl])
                    candv[sl] = nc
                    nst = top2_update(nst, nc, jnp.full((L,), j * L,
                                                        jnp.int32) + lanes)
                return nst

            bS1, bI1, bS2, bI2 = lax.fori_loop(
                0, NV // 4, sweep,
                (jnp.full((L,), NEG, jnp.float32), bigv,
                 jnp.full((L,), NEG, jnp.float32), bigv))

            @pl.when(accept & (sid == 0))
            def _():
                bvals = X1v
                bvals = jnp.where(lanes == 1, Y1v, bvals)
                bvals = jnp.where(lanes == 2, X2v, bvals)
                bvals = jnp.where(lanes == 3, Y2v, bvals)
                plsc.store_scatter(obv, [4 * k + lanes], bvals,
                                   mask=lanes < 4)
                kv = jnp.full((L,), k, jnp.int32)
                plsc.store_scatter(osv, [kv], jnp.full((L,), M, jnp.float32),
                                   mask=lanes == 0)
                plsc.store_scatter(olv, [kv], jnp.full((L,), wlb, jnp.int32),
                                   mask=lanes == 0)

            k = k + accept.astype(jnp.int32)
            return (accept, k, dead_a, dead_b, bS1, bI1, bS2, bI2)

        est = lax.while_loop(
            ex_cond, ex_body,
            (jnp.bool_(True), k0, dead_a0, dead_b0, bS1, bI1, bS2, bI2))
        k1 = est[1]
        cont = (k1 > k0) & (k1 < K)
        return (k1, cont, est[4], est[5], est[6], est[7])

    lax.while_loop(cond, body,
                   (jnp.int32(0), jnp.bool_(True)) + top0)

    @pl.when(sid == 0)
    def _():
        pltpu.sync_copy(obv, bo_hbm)
        pltpu.sync_copy(osv, so_hbm)
        pltpu.sync_copy(olv, lo_hbm)


@functools.partial(
    pl.kernel,
    out_type=(
        jax.ShapeDtypeStruct((BO_PAD,), jnp.float32),
        jax.ShapeDtypeStruct((SC_PAD,), jnp.float32),
        jax.ShapeDtypeStruct((SC_PAD,), jnp.int32),
    ),
    mesh=plsc.VectorSubcoreMesh(
        core_axis_name="c", subcore_axis_name="s",
        num_cores=1, num_subcores=NS),
    compiler_params=pltpu.CompilerParams(needs_layout_passes=False),
    scratch_types=[
        pltpu.VMEM((ROWS * C,), jnp.float32),
        pltpu.VMEM((ROWS,), jnp.float32),
        pltpu.VMEM((ROWS,), jnp.float32),
        pltpu.VMEM((ROWS,), jnp.float32),
        pltpu.VMEM((ROWS,), jnp.float32),
        pltpu.VMEM((ROWS,), jnp.float32),
        pltpu.VMEM((ROWS,), jnp.float32),
        pltpu.VMEM((ROWS,), jnp.int32),
        pltpu.VMEM((L,), jnp.float32),
        pltpu.VMEM((NS * L,), jnp.float32),
        pltpu.VMEM((BO_PAD,), jnp.float32),
        pltpu.VMEM((SC_PAD,), jnp.float32),
        pltpu.VMEM((SC_PAD,), jnp.int32),
        pltpu.VMEM_SHARED((2 * NS * L,), jnp.float32),
    ],
)
def _nms_call(sc_hbm, x1_hbm, y1_hbm, x2_hbm, y2_hbm,
              bo_hbm, so_hbm, lo_hbm, *scratch):
    _nms_kernel(sc_hbm, x1_hbm, y1_hbm, x2_hbm, y2_hbm,
                bo_hbm, so_hbm, lo_hbm, *scratch)


@jax.jit
def kernel(boxes, scores):
    bp = jnp.pad(boxes, ((0, NP - N), (0, 0)))
    bo, so, lo = _nms_call(
        scores.reshape(-1),
        bp[:, 0], bp[:, 1], bp[:, 2], bp[:, 3])
    return (bo[:4 * K].reshape(1, K, 4), so[:K][None], lo[:K][None])


# speedup vs baseline: 1.8393x; 1.6141x over previous
import functools

import jax
import jax.numpy as jnp
from jax import lax
from jax.experimental import pallas as pl
from jax.experimental.pallas import tpu as pltpu
from jax.experimental.pallas import tpu_sc as plsc

N = 5000
C = 80
THR = 0.2
IOU_THR = 0.2
K = 100

L = 16
NS = 16
ROWS = 320
NV = ROWS // L
NP = NS * ROWS
NEG = -1e30
BIG = 1 << 30
BO_PAD = 448
SC_PAD = 112


def _nms_kernel(sc_hbm, x1_hbm, y1_hbm, x2_hbm, y2_hbm,
                bo_hbm, so_hbm, lo_hbm,
                scv, x1v, y1v, x2v, y2v, arv, candv, labv,
                pubv, rbv, obv, osv, olv, shared):
    sid = lax.axis_index("s")
    base = sid * ROWS
    lanes = lax.iota(jnp.int32, L)
    zf = jnp.zeros((L,), jnp.float32)
    zi = jnp.zeros((L,), jnp.int32)
    negv = jnp.full((L,), NEG, jnp.float32)
    bigv = jnp.full((L,), BIG, jnp.int32)

    @pl.when(sid < NS - 1)
    def _():
        pltpu.sync_copy(sc_hbm.at[pl.ds(base * C, ROWS * C)], scv)

    @pl.when(sid == NS - 1)
    def _():
        pltpu.sync_copy(sc_hbm.at[pl.ds(base * C, (N - (NS - 1) * ROWS) * C)],
                        scv.at[pl.ds(0, (N - (NS - 1) * ROWS) * C)])

    pltpu.sync_copy(x1_hbm.at[pl.ds(base, ROWS)], x1v)
    pltpu.sync_copy(y1_hbm.at[pl.ds(base, ROWS)], y1v)
    pltpu.sync_copy(x2_hbm.at[pl.ds(base, ROWS)], x2v)
    pltpu.sync_copy(y2_hbm.at[pl.ds(base, ROWS)], y2v)

    def zero_b(j, _):
        obv[pl.ds(j * L, L)] = zf
        return 0
    lax.fori_loop(0, BO_PAD // L, zero_b, 0)

    def zero_s(j, _):
        osv[pl.ds(j * L, L)] = zf
        olv[pl.ds(j * L, L)] = zi
        return 0
    lax.fori_loop(0, SC_PAD // L, zero_s, 0)

    def top2_update(st, v, idx):
        bS1, bI1, bS2, bI2 = st
        take1 = v > bS1
        take2 = (v > bS2) & (~take1)
        nS2 = jnp.where(take1, bS1, jnp.where(take2, v, bS2))
        nI2 = jnp.where(take1, bI1, jnp.where(take2, idx, bI2))
        nS1 = jnp.where(take1, v, bS1)
        nI1 = jnp.where(take1, idx, bI1)
        return (nS1, nI1, nS2, nI2)

    def grp(g, _):
        row_off = g * (L * C) + lanes * C

        def cls(c8, MA):
            M, A = MA
            for dc in range(8):
                c = c8 * 8 + dc
                v = plsc.load_gather(scv, [row_off + c])
                take = v > M
                M = jnp.where(take, v, M)
                A = jnp.where(take, jnp.full((L,), c, jnp.int32), A)
            return (M, A)

        M, A = lax.fori_loop(0, C // 8, cls,
                             (jnp.full((L,), NEG, jnp.float32), zi))
        sl = pl.ds(g * L, L)
        validrow = (base + g * L + lanes) < N
        candv[sl] = jnp.where(validrow & (M > THR), M, negv)
        labv[sl] = A
        arv[sl] = (x2v[sl] - x1v[sl]) * (y2v[sl] - y1v[sl])
        return 0
    lax.fori_loop(0, NV, grp, 0)

    def cond(st):
        return st[1]

    def body(st):
        k0, _ = st

        def scan(j2, nst):
            for dj in range(2):
                j = j2 * 2 + dj
                nst = top2_update(nst, candv[pl.ds(j * L, L)],
                                  jnp.full((L,), j * L, jnp.int32) + lanes)
            return nst

        bS1, bI1, bS2, bI2 = lax.fori_loop(
            0, NV // 2, scan,
            (jnp.full((L,), NEG, jnp.float32), bigv,
             jnp.full((L,), NEG, jnp.float32), bigv))

        M1 = jnp.max(bS1)
        li1 = jnp.min(jnp.where(bS1 == M1, bI1, bigv))
        match = (bS1 == M1) & (bI1 == li1)
        bS1m = jnp.where(match, bS2, bS1)
        bI1m = jnp.where(match, bI2, bI1)
        M2 = jnp.max(bS1m)
        li2 = jnp.min(jnp.where(bS1m == M2, bI1m, bigv))
        li1c = jnp.clip(li1, 0, ROWS - 1)
        li2c = jnp.clip(li2, 0, ROWS - 1)

        l1v = jnp.full((L,), li1c, jnp.int32)
        l2v = jnp.full((L,), li2c, jnp.int32)
        a_x1 = plsc.load_gather(x1v, [l1v])
        a_y1 = plsc.load_gather(y1v, [l1v])
        a_x2 = plsc.load_gather(x2v, [l1v])
        a_y2 = plsc.load_gather(y2v, [l1v])
        a_ar = plsc.load_gather(arv, [l1v])
        a_lb = plsc.load_gather(labv, [l1v])
        b_x1 = plsc.load_gather(x1v, [l2v])
        b_y1 = plsc.load_gather(y1v, [l2v])
        b_x2 = plsc.load_gather(x2v, [l2v])
        b_y2 = plsc.load_gather(y2v, [l2v])
        b_ar = plsc.load_gather(arv, [l2v])
        b_lb = plsc.load_gather(labv, [l2v])

        pub = jnp.full((L,), M1, jnp.float32)
        pk1 = jnp.full((L,), sid * 65536 + li1c, jnp.int32)
        pk2 = jnp.full((L,), sid * 65536 + li2c, jnp.int32)
        pub = jnp.where(lanes == 1, plsc.bitcast(pk1, jnp.float32), pub)
        pub = jnp.where(lanes == 2, a_x1, pub)
        pub = jnp.where(lanes == 3, a_y1, pub)
        pub = jnp.where(lanes == 4, a_x2, pub)
        pub = jnp.where(lanes == 5, a_y2, pub)
        pub = jnp.where(lanes == 6, plsc.bitcast(a_lb, jnp.float32), pub)
        pub = jnp.where(lanes == 7, a_ar, pub)
        pub = jnp.where(lanes == 8, jnp.full((L,), M2, jnp.float32), pub)
        pub = jnp.where(lanes == 9, plsc.bitcast(pk2, jnp.float32), pub)
        pub = jnp.where(lanes == 10, b_x1, pub)
        pub = jnp.where(lanes == 11, b_y1, pub)
        pub = jnp.where(lanes == 12, b_x2, pub)
        pub = jnp.where(lanes == 13, b_y2, pub)
        pub = jnp.where(lanes == 14, plsc.bitcast(b_lb, jnp.float32), pub)
        pub = jnp.where(lanes == 15, b_ar, pub)
        pubv[...] = pub

        par = lax.rem(k0, 2)
        pltpu.sync_copy(pubv, shared.at[pl.ds(par * (NS * L) + sid * L, L)])
        plsc.subcore_barrier()
        pltpu.sync_copy(shared.at[pl.ds(par * (NS * L), NS * L)], rbv)

        def col(c):
            return plsc.load_gather(rbv, [lanes * L + c])

        s_a = col(0)
        p_a = plsc.bitcast(col(1), jnp.int32)
        ax1 = col(2)
        ay1 = col(3)
        ax2 = col(4)
        ay2 = col(5)
        aar = col(7)
        s_b = col(8)
        p_b = plsc.bitcast(col(9), jnp.int32)
        bx1 = col(10)
        by1 = col(11)
        bx2 = col(12)
        by2 = col(13)
        bar = col(15)

        dead_a0 = ~(s_a > -1e29)
        dead_b0 = ~(s_b > -1e29)

        def ex_cond(est):
            return est[0]

        def ex_body(est):
            _, k, dead_a, dead_b = est

            ma = jnp.where(dead_a, negv, s_a)
            mb = jnp.where(dead_b, negv, s_b)
            M = jnp.max(jnp.maximum(ma, mb))
            Mv = jnp.full((L,), M, jnp.float32)
            pa = jnp.where((~dead_a) & (s_a == Mv), p_a, bigv)
            pb = jnp.where((~dead_b) & (s_b == Mv), p_b, bigv)
            pw = jnp.min(jnp.minimum(pa, pb))
            pwv = jnp.full((L,), pw, jnp.int32)
            got = M > -1e29
            bothdead = dead_a & dead_b
            outrank = (s_b > Mv) | ((s_b == Mv) & (p_b < pwv))
            viol = jnp.max((bothdead & outrank).astype(jnp.int32)) > 0
            accept = got & (~viol) & (k < K)

            is_w_a = (~dead_a) & (s_a == Mv) & (p_a == pwv)
            is_w_b = (~dead_b) & (s_b == Mv) & (p_b == pwv)
            from_b = jnp.max(is_w_b.astype(jnp.int32)) > 0

            wsid = jnp.clip(jnp.right_shift(pw, 16), 0, NS - 1)
            srow = wsid * L + jnp.where(from_b, 8, 0)
            sr = jnp.full((L,), srow, jnp.int32)
            X1v = plsc.load_gather(rbv, [sr + 2])
            Y1v = plsc.load_gather(rbv, [sr + 3])
            X2v = plsc.load_gather(rbv, [sr + 4])
            Y2v = plsc.load_gather(rbv, [sr + 5])
            LBv = plsc.bitcast(plsc.load_gather(rbv, [sr + 6]), jnp.int32)
            WAv = plsc.load_gather(rbv, [sr + 7])
            iwv = jnp.full((L,), (pw & 0xFFFF) + wsid * ROWS, jnp.int32)
            acc_v = jnp.full((L,), accept, jnp.bool_)

            def board_iou(ex1, ey1, ex2, ey2, ear):
                xx1 = jnp.maximum(ex1, X1v)
                yy1 = jnp.maximum(ey1, Y1v)
                xx2 = jnp.minimum(ex2, X2v)
                yy2 = jnp.minimum(ey2, Y2v)
                inter = (jnp.maximum(xx2 - xx1, 0.0) *
                         jnp.maximum(yy2 - yy1, 0.0))
                union = WAv + ear - inter
                return inter / (union + 1e-8)

            kill_a = acc_v & (is_w_a | (board_iou(ax1, ay1, ax2, ay2, aar)
                                        >= IOU_THR))
            kill_b = acc_v & (is_w_b | (board_iou(bx1, by1, bx2, by2, bar)
                                        >= IOU_THR))
            dead_a = dead_a | kill_a
            dead_b = dead_b | kill_b

            @pl.when(accept)
            def _():
                def sweep(j4, _):
                    for dj in range(4):
                        j = j4 * 4 + dj
                        sl = pl.ds(j * L, L)
                        xx1 = jnp.maximum(x1v[sl], X1v)
                        yy1 = jnp.maximum(y1v[sl], Y1v)
                        xx2 = jnp.minimum(x2v[sl], X2v)
                        yy2 = jnp.minimum(y2v[sl], Y2v)
                        inter = (jnp.maximum(xx2 - xx1, 0.0) *
                                 jnp.maximum(yy2 - yy1, 0.0))
                        union = WAv + arv[sl] - inter
                        iou = inter / (union + 1e-8)
                        gi = base + j * L + lanes
                        kill = (iou >= IOU_THR) | (gi == iwv)
                        candv[sl] = jnp.where(kill, negv, candv[sl])
                    return 0
                lax.fori_loop(0, NV // 4, sweep, 0)

            @pl.when(accept & (sid == 0))
            def _():
                bvals = X1v
                bvals = jnp.where(lanes == 1, Y1v, bvals)
                bvals = jnp.where(lanes == 2, X2v, bvals)
                bvals = jnp.where(lanes == 3, Y2v, bvals)
                plsc.store_scatter(obv, [4 * k + lanes], bvals,
                                   mask=lanes < 4)
                kv = jnp.full((L,), k, jnp.int32)
                plsc.store_scatter(osv, [kv], jnp.full((L,), M, jnp.float32),
                                   mask=lanes == 0)
                plsc.store_scatter(olv, [kv], LBv, mask=lanes == 0)

            k = k + accept.astype(jnp.int32)
            return (accept, k, dead_a, dead_b)

        est = lax.while_loop(ex_cond, ex_body,
                             (jnp.bool_(True), k0, dead_a0, dead_b0))
        k1 = est[1]
        cont = (k1 > k0) & (k1 < K)
        return (k1, cont)

    lax.while_loop(cond, body, (jnp.int32(0), jnp.bool_(True)))

    @pl.when(sid == 0)
    def _():
        pltpu.sync_copy(obv, bo_hbm)
        pltpu.sync_copy(osv, so_hbm)
        pltpu.sync_copy(olv, lo_hbm)


@functools.partial(
    pl.kernel,
    out_type=(
        jax.ShapeDtypeStruct((BO_PAD,), jnp.float32),
        jax.ShapeDtypeStruct((SC_PAD,), jnp.float32),
        jax.ShapeDtypeStruct((SC_PAD,), jnp.int32),
    ),
    mesh=plsc.VectorSubcoreMesh(
        core_axis_name="c", subcore_axis_name="s",
        num_cores=1, num_subcores=NS),
    compiler_params=pltpu.CompilerParams(needs_layout_passes=False),
    scratch_types=[
        pltpu.VMEM((ROWS * C,), jnp.float32),
        pltpu.VMEM((ROWS,), jnp.float32),
        pltpu.VMEM((ROWS,), jnp.float32),
        pltpu.VMEM((ROWS,), jnp.float32),
        pltpu.VMEM((ROWS,), jnp.float32),
        pltpu.VMEM((ROWS,), jnp.float32),
        pltpu.VMEM((ROWS,), jnp.float32),
        pltpu.VMEM((ROWS,), jnp.int32),
        pltpu.VMEM((L,), jnp.float32),
        pltpu.VMEM((NS * L,), jnp.float32),
        pltpu.VMEM((BO_PAD,), jnp.float32),
        pltpu.VMEM((SC_PAD,), jnp.float32),
        pltpu.VMEM((SC_PAD,), jnp.int32),
        pltpu.VMEM_SHARED((2 * NS * L,), jnp.float32),
    ],
)
def _nms_call(sc_hbm, x1_hbm, y1_hbm, x2_hbm, y2_hbm,
              bo_hbm, so_hbm, lo_hbm, *scratch):
    _nms_kernel(sc_hbm, x1_hbm, y1_hbm, x2_hbm, y2_hbm,
                bo_hbm, so_hbm, lo_hbm, *scratch)


@jax.jit
def kernel(boxes, scores):
    bp = jnp.pad(boxes, ((0, NP - N), (0, 0)))
    bo, so, lo = _nms_call(
        scores.reshape(-1),
        bp[:, 0], bp[:, 1], bp[:, 2], bp[:, 3])
    return (bo[:4 * K].reshape(1, K, 4), so[:K][None], lo[:K][None])
